# Initial kernel scaffold; baseline (speedup 1.0000x reference)
import jax
import jax.numpy as jnp
from jax.experimental import pallas as pl


def kernel(points, features, batch, W_enc, b_enc, W1, b1, W2, b2, W3, b3, Wg1, bg1, Wg2, bg2, Wg3, bg3):
    raise NotImplementedError("write your pallas kernel here")


# speedup vs baseline: 4.8014x; 4.8014x over previous
import functools

import jax
import jax.numpy as jnp
from jax import lax
from jax.experimental import pallas as pl
from jax.experimental.pallas import tpu as pltpu
from jax.experimental.pallas import tpu_sc as plsc

N = 10000
NB = 16
M = N // NB
MP = 640
K = 32
RADIUS = 0.2
R2 = float(RADIUS * RADIUS)
NEG = -1e9

FPS_S = 8
FPS_L = N // FPS_S

SC_NC = 2
SC_NS = 16
SC_NW = SC_NC * SC_NS
E = MP * K
E_PER_W = E // SC_NW
GCHUNK = 128

PD = 16


def _fps_body(xs_ref, ys_ref, zs_ref, idx_ref):
    iota = (lax.broadcasted_iota(jnp.int32, (FPS_S, FPS_L), 0) * FPS_L
            + lax.broadcasted_iota(jnp.int32, (FPS_S, FPS_L), 1))
    xs = xs_ref[...]
    ys = ys_ref[...]
    zs = zs_ref[...]
    idx_ref[0] = 0

    def body(i, carry):
        mind, last = carry
        lmask = iota == last
        lx = jnp.sum(jnp.where(lmask, xs, 0.0))
        ly = jnp.sum(jnp.where(lmask, ys, 0.0))
        lz = jnp.sum(jnp.where(lmask, zs, 0.0))
        dx = xs - lx
        dy = ys - ly
        dz = zs - lz
        d = (dx * dx + dy * dy) + dz * dz
        mind = jnp.minimum(mind, d)
        mx = jnp.max(mind)
        nxt = jnp.min(jnp.where(mind == mx, iota, jnp.int32(N)))
        idx_ref[i] = nxt
        return (mind, nxt)

    mind0 = jnp.full((FPS_S, FPS_L), jnp.inf, jnp.float32)
    lax.fori_loop(1, M, body, (mind0, jnp.int32(0)))


def _fps(xs, ys, zs, interpret=False):
    return pl.pallas_call(
        _fps_body,
        out_shape=jax.ShapeDtypeStruct((M,), jnp.int32),
        in_specs=[pl.BlockSpec(memory_space=pltpu.VMEM)] * 3,
        out_specs=pl.BlockSpec(memory_space=pltpu.SMEM),
        interpret=interpret,
    )(xs, ys, zs)


def _topk_body(ptsT_ref, fpsp_ref, idx_ref, d2_ref):
    px = ptsT_ref[0:1, :]
    py = ptsT_ref[1:2, :]
    pz = ptsT_ref[2:3, :]
    fx = fpsp_ref[:, 0:1]
    fy = fpsp_ref[:, 1:2]
    fz = fpsp_ref[:, 2:3]
    dx = fx - px
    dy = fy - py
    dz = fz - pz
    d = (dx * dx + dy * dy) + dz * dz
    iota = lax.broadcasted_iota(jnp.int32, (8, N), 1)
    for j in range(K):
        m = jnp.min(d, axis=1, keepdims=True)
        sel = jnp.min(jnp.where(d == m, iota, jnp.int32(N)),
                      axis=1, keepdims=True)
        idx_ref[:, j:j + 1] = sel
        d2_ref[:, j:j + 1] = m
        d = jnp.where(iota == sel, jnp.inf, d)


def _topk(ptsT, fpsp_pad, interpret=False):
    grid = MP // 8
    return pl.pallas_call(
        _topk_body,
        grid=(grid,),
        in_specs=[
            pl.BlockSpec((3, N), lambda g: (0, 0)),
            pl.BlockSpec((8, 3), lambda g: (g, 0)),
        ],
        out_specs=[
            pl.BlockSpec((8, K), lambda g: (g, 0)),
            pl.BlockSpec((8, K), lambda g: (g, 0)),
        ],
        out_shape=[
            jax.ShapeDtypeStruct((MP, K), jnp.int32),
            jax.ShapeDtypeStruct((MP, K), jnp.float32),
        ],
        interpret=interpret,
    )(ptsT, fpsp_pad)


def _gather_body(x_hbm, y_hbm, z_hbm, feat_hbm, idx_hbm,
                 outx_hbm, outy_hbm, outz_hbm, outf_hbm,
                 x_v, y_v, z_v, idx_v, px, py, pz, rows_f, sem):
    wid = lax.axis_index("s") * SC_NC + lax.axis_index("c")
    base = wid * E_PER_W
    pltpu.sync_copy(idx_hbm.at[pl.ds(base, E_PER_W)], idx_v)
    copies = []
    for j in range(0, E_PER_W, GCHUNK):
        cnt = min(GCHUNK, E_PER_W - j)
        sl = pl.ds(j, cnt)
        copies.append(pltpu.async_copy(
            feat_hbm.at[idx_v.at[sl]], rows_f.at[sl], sem))
    pltpu.sync_copy(x_hbm, x_v)
    pltpu.sync_copy(y_hbm, y_v)
    pltpu.sync_copy(z_hbm, z_v)
    for j in range(0, E_PER_W, 16):
        sl = pl.ds(j, 16)
        iv = idx_v[sl]
        px[sl] = plsc.load_gather(x_v, [iv])
        py[sl] = plsc.load_gather(y_v, [iv])
        pz[sl] = plsc.load_gather(z_v, [iv])
    for c in copies:
        c.wait()
    pltpu.sync_copy(px, outx_hbm.at[pl.ds(base, E_PER_W)])
    pltpu.sync_copy(py, outy_hbm.at[pl.ds(base, E_PER_W)])
    pltpu.sync_copy(pz, outz_hbm.at[pl.ds(base, E_PER_W)])
    pltpu.sync_copy(rows_f, outf_hbm.at[pl.ds(base, E_PER_W)])


def _sc_gather(x, y, z, features, idx_flat):
    mesh = plsc.VectorSubcoreMesh(core_axis_name="c", subcore_axis_name="s")
    kfn = pl.kernel(
        _gather_body,
        out_type=[
            jax.ShapeDtypeStruct((E,), jnp.float32),
            jax.ShapeDtypeStruct((E,), jnp.float32),
            jax.ShapeDtypeStruct((E,), jnp.float32),
            jax.ShapeDtypeStruct((E, 128), jnp.float32),
        ],
        mesh=mesh,
        scratch_types=[
            pltpu.VMEM((N,), jnp.float32),
            pltpu.VMEM((N,), jnp.float32),
            pltpu.VMEM((N,), jnp.float32),
            pltpu.VMEM((E_PER_W,), jnp.int32),
            pltpu.VMEM((E_PER_W,), jnp.float32),
            pltpu.VMEM((E_PER_W,), jnp.float32),
            pltpu.VMEM((E_PER_W,), jnp.float32),
            pltpu.VMEM((E_PER_W, 128), jnp.float32),
            pltpu.SemaphoreType.DMA,
        ],
        compiler_params=pltpu.CompilerParams(needs_layout_passes=False),
    )
    return kfn(x, y, z, features, idx_flat)


CB = 160


def _edge_body(nbrx_ref, nbry_ref, nbrz_ref, nbrf_ref, fpsp_ref, d2_ref,
               Wenc_ref, benc_ref, W1a_ref, W1b_ref, W1c_ref, b1_ref,
               W2_ref, b2_ref, W3_ref, b3_ref,
               renc_ref, maxf_ref):
    inv_r = jnp.float32(1.0) / jnp.float32(RADIUS)
    relx = (nbrx_ref[...] - fpsp_ref[:, 0:1, :]) * inv_r
    rely = (nbry_ref[...] - fpsp_ref[:, 1:2, :]) * inv_r
    relz = (nbrz_ref[...] - fpsp_ref[:, 2:3, :]) * inv_r
    valid_f = jnp.where(d2_ref[...] <= R2, 1.0, 0.0)
    penal = (1.0 - valid_f) * NEG

    def mask_max(x3):
        return jnp.max(x3 * valid_f + penal, axis=1)

    def small_mm(W_ref, b):
        out = (relx * W_ref[0:1, :][None]
               + rely * W_ref[1:2, :][None]
               + relz * W_ref[2:3, :][None])
        return out + b[None]

    h3 = jax.nn.relu(small_mm(Wenc_ref, benc_ref[...]))
    renc = mask_max(h3)
    renc_ref[...] = renc

    relm = jnp.broadcast_to(renc[:, None, :], (CB, K, 32)).reshape(CB * K, 32)
    nbrf = nbrf_ref[...].reshape(CB * K, 128)
    f1a = small_mm(W1a_ref, b1_ref[...]).reshape(CB * K, 64)
    f1 = jax.nn.relu(
        f1a
        + jnp.dot(relm, W1b_ref[...], preferred_element_type=jnp.float32)
        + jnp.dot(nbrf, W1c_ref[...], preferred_element_type=jnp.float32))
    f2 = jax.nn.relu(
        jnp.dot(f1, W2_ref[...], preferred_element_type=jnp.float32)
        + b2_ref[...])
    f3 = jax.nn.relu(
        jnp.dot(f2, W3_ref[...], preferred_element_type=jnp.float32)
        + b3_ref[...])
    maxf_ref[...] = mask_max(f3.reshape(CB, K, 128))


def _edge(nbrx, nbry, nbrz, nbrf3, fpsp, d2sel, Wenc, benc, W1a, W1b, W1c,
          b1, W2, b2, W3, b3, interpret=False):
    grid = MP // CB
    full = lambda shape: pl.BlockSpec(shape, lambda g: tuple(0 for _ in shape))
    return pl.pallas_call(
        _edge_body,
        grid=(grid,),
        in_specs=[
            pl.BlockSpec((CB, K, 1), lambda g: (g, 0, 0)),
            pl.BlockSpec((CB, K, 1), lambda g: (g, 0, 0)),
            pl.BlockSpec((CB, K, 1), lambda g: (g, 0, 0)),
            pl.BlockSpec((CB, K, 128), lambda g: (g, 0, 0)),
            pl.BlockSpec((CB, 3, 1), lambda g: (g, 0, 0)),
            pl.BlockSpec((CB, K, 1), lambda g: (g, 0, 0)),
            full((3, 32)), full((1, 32)),
            full((3, 64)), full((32, 64)), full((128, 64)), full((1, 64)),
            full((64, 64)), full((1, 64)),
            full((64, 128)), full((1, 128)),
        ],
        out_specs=[
            pl.BlockSpec((CB, 32), lambda g: (g, 0)),
            pl.BlockSpec((CB, 128), lambda g: (g, 0)),
        ],
        out_shape=[
            jax.ShapeDtypeStruct((MP, 32), jnp.float32),
            jax.ShapeDtypeStruct((MP, 128), jnp.float32),
        ],
        interpret=interpret,
    )(nbrx, nbry, nbrz, nbrf3, fpsp, d2sel, Wenc, benc, W1a, W1b, W1c, b1,
      W2, b2, W3, b3)


def _gmlp_body(maxf_ref, Wg1_ref, bg1_ref, Wg2_ref, bg2_ref,
               Wg3_ref, bg3_ref, out_ref):
    g1 = jax.nn.relu(
        jnp.dot(maxf_ref[...], Wg1_ref[...],
                preferred_element_type=jnp.float32) + bg1_ref[...])
    g2 = jax.nn.relu(
        jnp.dot(g1, Wg2_ref[...], preferred_element_type=jnp.float32)
        + bg2_ref[...])
    out_ref[...] = jax.nn.relu(
        jnp.dot(g2, Wg3_ref[...], preferred_element_type=jnp.float32)
        + bg3_ref[...])


def _gmlp(maxf, Wg1, bg1, Wg2, bg2, Wg3, bg3, interpret=False):
    return pl.pallas_call(
        _gmlp_body,
        out_shape=jax.ShapeDtypeStruct((MP, 64), jnp.float32),
        interpret=interpret,
    )(maxf, Wg1, bg1, Wg2, bg2, Wg3, bg3)


@jax.jit
def _run(points, features, batch, W_enc, b_enc, W1, b1, W2, b2, W3, b3,
         Wg1, bg1, Wg2, bg2, Wg3, bg3):
    ptsT = points.T
    xs = ptsT[0].reshape(FPS_S, FPS_L)
    ys = ptsT[1].reshape(FPS_S, FPS_L)
    zs = ptsT[2].reshape(FPS_S, FPS_L)
    fps_idx = _fps(xs, ys, zs)

    fps_idx_pad = jnp.concatenate(
        [fps_idx, jnp.zeros((MP - M,), jnp.int32)])
    fpsp_pad = points[fps_idx_pad]
    nbr_idx, d2sel = _topk(ptsT, fpsp_pad)

    idx_flat = nbr_idx.reshape(-1)
    outx, outy, outz, outf = _sc_gather(
        points[:, 0], points[:, 1], points[:, 2], features, idx_flat)
    nbrx = outx.reshape(MP, K, 1)
    nbry = outy.reshape(MP, K, 1)
    nbrz = outz.reshape(MP, K, 1)
    nbrf3 = outf.reshape(MP, K, 128)

    renc, maxf = _edge(
        nbrx, nbry, nbrz, nbrf3, fpsp_pad.reshape(MP, 3, 1),
        d2sel.reshape(MP, K, 1), W_enc,
        b_enc.reshape(1, -1), W1[:3], W1[3:35], W1[35:],
        b1.reshape(1, -1), W2, b2.reshape(1, -1), W3, b3.reshape(1, -1))

    g3 = _gmlp(maxf, Wg1, bg1.reshape(1, -1), Wg2, bg2.reshape(1, -1),
               Wg3, bg3.reshape(1, -1))

    fps_points = fpsp_pad[:M]
    output_features = jnp.concatenate([renc[:M], g3[:M]], axis=1)
    fps_batch = batch[fps_idx]
    return (fps_points, output_features, fps_batch)


def kernel(points, features, batch, W_enc, b_enc, W1, b1, W2, b2, W3, b3,
           Wg1, bg1, Wg2, bg2, Wg3, bg3):
    return _run(points, features, batch, W_enc, b_enc, W1, b1, W2, b2,
                W3, b3, Wg1, bg1, Wg2, bg2, Wg3, bg3)


# speedup vs baseline: 7.1334x; 1.4857x over previous
import functools

import jax
import jax.numpy as jnp
from jax import lax
from jax.experimental import pallas as pl
from jax.experimental.pallas import tpu as pltpu
from jax.experimental.pallas import tpu_sc as plsc

N = 10000
NB = 16
M = N // NB
MP = 640
K = 32
RADIUS = 0.2
R2 = float(RADIUS * RADIUS)
NEG = -1e9

FPS_S = 8
FPS_L = N // FPS_S

SC_NC = 2
SC_NS = 16
SC_NW = SC_NC * SC_NS
E = MP * K
E_PER_W = E // SC_NW
GCHUNK = 128

PD = 16


FPS_LP = 1280
FPS_NCH = FPS_LP // 128


def _argmax2(m, i):
    ms = [m[:, c * 128:(c + 1) * 128] for c in range(FPS_NCH)]
    is_ = [i[:, c * 128:(c + 1) * 128] for c in range(FPS_NCH)]
    while len(ms) > 1:
        h = len(ms) // 2
        nm, ni = [], []
        for a in range(h):
            b = a + h
            t = ms[b] > ms[a]
            nm.append(jnp.where(t, ms[b], ms[a]))
            ni.append(jnp.where(t, is_[b], is_[a]))
        if len(ms) % 2:
            nm.append(ms[2 * h]); ni.append(is_[2 * h])
        ms, is_ = nm, ni
    wm, wi = ms[0], is_[0]
    for axis, shifts in ((1, (64, 32, 16, 8, 4, 2, 1)), (0, (4, 2, 1))):
        for sh in shifts:
            om = pltpu.roll(wm, sh, axis)
            oi = pltpu.roll(wi, sh, axis)
            t = (om > wm) | ((om == wm) & (oi < wi))
            wm = jnp.where(t, om, wm)
            wi = jnp.where(t, oi, wi)
    return wm, wi


def _fps_body(xs_ref, ys_ref, zs_ref, xsm_ref, ysm_ref, zsm_ref, idx_ref):
    lane = lax.broadcasted_iota(jnp.int32, (FPS_S, FPS_LP), 1)
    sub = lax.broadcasted_iota(jnp.int32, (FPS_S, FPS_LP), 0)
    valid = lane < FPS_L
    iota = jnp.where(valid, sub * FPS_L + lane, jnp.int32(N))
    xs = xs_ref[...]
    ys = ys_ref[...]
    zs = zs_ref[...]
    idx_ref[0] = 0
    mind0 = jnp.where(valid, jnp.inf, jnp.float32(-jnp.inf))

    def body(i, carry):
        mind, last = carry
        bx = xsm_ref[last]
        by = ysm_ref[last]
        bz = zsm_ref[last]
        dx = xs - bx
        dy = ys - by
        dz = zs - bz
        d = (dx * dx + dy * dy) + dz * dz
        mind = jnp.minimum(mind, d)
        mx = jnp.max(mind)
        nxt = jnp.min(jnp.where(mind == mx, iota, jnp.int32(N)))
        idx_ref[i] = nxt
        return (mind, nxt)

    lax.fori_loop(1, M, body, (mind0, jnp.int32(0)))


def _fps(xs, ys, zs, xf, yf, zf, interpret=False):
    return pl.pallas_call(
        _fps_body,
        out_shape=jax.ShapeDtypeStruct((M,), jnp.int32),
        in_specs=[pl.BlockSpec(memory_space=pltpu.VMEM)] * 3
        + [pl.BlockSpec(memory_space=pltpu.SMEM)] * 3,
        out_specs=pl.BlockSpec(memory_space=pltpu.SMEM),
        interpret=interpret,
    )(xs, ys, zs, xf, yf, zf)


NP = 10240
NCH = NP // 128
CBK = 32


def _pair_tree(ds, is_):
    while len(ds) > 1:
        h = len(ds) // 2
        nd, ni = [], []
        for a in range(h):
            b = a + h
            take_b = ds[b] < ds[a]
            nd.append(jnp.where(take_b, ds[b], ds[a]))
            ni.append(jnp.where(take_b, is_[b], is_[a]))
        if len(ds) % 2:
            nd.append(ds[2 * h])
            ni.append(is_[2 * h])
        ds, is_ = nd, ni
    return ds[0], is_[0]


def _lane_fold(wd, wi):
    for sh in (64, 32, 16, 8, 4, 2, 1):
        od = pltpu.roll(wd, sh, 1)
        oi = pltpu.roll(wi, sh, 1)
        take = (od < wd) | ((od == wd) & (oi < wi))
        wd = jnp.where(take, od, wd)
        wi = jnp.where(take, oi, wi)
    return wd, wi


def _topk_body(ptsT_ref, fpsp_ref, idx_ref, d2_ref):
    fx = fpsp_ref[:, 0:1]
    fy = fpsp_ref[:, 1:2]
    fz = fpsp_ref[:, 2:3]
    dch = []
    ich = []
    for c in range(NCH):
        sl = pl.ds(c * 128, 128)
        dx = fx - ptsT_ref[0:1, sl]
        dy = fy - ptsT_ref[1:2, sl]
        dz = fz - ptsT_ref[2:3, sl]
        dch.append((dx * dx + dy * dy) + dz * dz)
        ich.append(lax.broadcasted_iota(jnp.int32, (CBK, 128), 1) + c * 128)
    for j in range(K):
        cd, ci = _pair_tree(list(dch), list(ich))
        wd, wi = _lane_fold(cd, ci)
        idx_ref[:, j:j + 1] = wi[:, 0:1]
        d2_ref[:, j:j + 1] = wd[:, 0:1]
        for c in range(NCH):
            dch[c] = jnp.where(ich[c] == wi, jnp.inf, dch[c])


def _topk(ptsT_pad, fpsp_pad, interpret=False):
    grid = MP // CBK
    return pl.pallas_call(
        _topk_body,
        grid=(grid,),
        in_specs=[
            pl.BlockSpec((3, NP), lambda g: (0, 0)),
            pl.BlockSpec((CBK, 3), lambda g: (g, 0)),
        ],
        out_specs=[
            pl.BlockSpec((CBK, K), lambda g: (g, 0)),
            pl.BlockSpec((CBK, K), lambda g: (g, 0)),
        ],
        out_shape=[
            jax.ShapeDtypeStruct((MP, K), jnp.int32),
            jax.ShapeDtypeStruct((MP, K), jnp.float32),
        ],
        interpret=interpret,
    )(ptsT_pad, fpsp_pad)


def _gather_body(x_hbm, y_hbm, z_hbm, feat_hbm, idx_hbm,
                 outx_hbm, outy_hbm, outz_hbm, outf_hbm,
                 x_v, y_v, z_v, idx_v, px, py, pz, rows_f, sem):
    wid = lax.axis_index("s") * SC_NC + lax.axis_index("c")
    base = wid * E_PER_W
    pltpu.sync_copy(idx_hbm.at[pl.ds(base, E_PER_W)], idx_v)
    copies = []
    for j in range(0, E_PER_W, GCHUNK):
        cnt = min(GCHUNK, E_PER_W - j)
        sl = pl.ds(j, cnt)
        copies.append(pltpu.async_copy(
            feat_hbm.at[idx_v.at[sl]], rows_f.at[sl], sem))
    pltpu.sync_copy(x_hbm, x_v)
    pltpu.sync_copy(y_hbm, y_v)
    pltpu.sync_copy(z_hbm, z_v)
    for j in range(0, E_PER_W, 16):
        sl = pl.ds(j, 16)
        iv = idx_v[sl]
        px[sl] = plsc.load_gather(x_v, [iv])
        py[sl] = plsc.load_gather(y_v, [iv])
        pz[sl] = plsc.load_gather(z_v, [iv])
    for c in copies:
        c.wait()
    pltpu.sync_copy(px, outx_hbm.at[pl.ds(base, E_PER_W)])
    pltpu.sync_copy(py, outy_hbm.at[pl.ds(base, E_PER_W)])
    pltpu.sync_copy(pz, outz_hbm.at[pl.ds(base, E_PER_W)])
    pltpu.sync_copy(rows_f, outf_hbm.at[pl.ds(base, E_PER_W)])


def _sc_gather(x, y, z, features, idx_flat):
    mesh = plsc.VectorSubcoreMesh(core_axis_name="c", subcore_axis_name="s")
    kfn = pl.kernel(
        _gather_body,
        out_type=[
            jax.ShapeDtypeStruct((E,), jnp.float32),
            jax.ShapeDtypeStruct((E,), jnp.float32),
            jax.ShapeDtypeStruct((E,), jnp.float32),
            jax.ShapeDtypeStruct((E, 128), jnp.float32),
        ],
        mesh=mesh,
        scratch_types=[
            pltpu.VMEM((N,), jnp.float32),
            pltpu.VMEM((N,), jnp.float32),
            pltpu.VMEM((N,), jnp.float32),
            pltpu.VMEM((E_PER_W,), jnp.int32),
            pltpu.VMEM((E_PER_W,), jnp.float32),
            pltpu.VMEM((E_PER_W,), jnp.float32),
            pltpu.VMEM((E_PER_W,), jnp.float32),
            pltpu.VMEM((E_PER_W, 128), jnp.float32),
            pltpu.SemaphoreType.DMA,
        ],
        compiler_params=pltpu.CompilerParams(needs_layout_passes=False),
    )
    return kfn(x, y, z, features, idx_flat)


CB = 160


def _edge_body(nbrx_ref, nbry_ref, nbrz_ref, nbrf_ref, fpsp_ref, d2_ref,
               Wenc_ref, benc_ref, W1a_ref, W1b_ref, W1c_ref, b1_ref,
               W2_ref, b2_ref, W3_ref, b3_ref,
               renc_ref, maxf_ref):
    inv_r = jnp.float32(1.0) / jnp.float32(RADIUS)
    relx = (nbrx_ref[...] - fpsp_ref[:, 0:1, :]) * inv_r
    rely = (nbry_ref[...] - fpsp_ref[:, 1:2, :]) * inv_r
    relz = (nbrz_ref[...] - fpsp_ref[:, 2:3, :]) * inv_r
    valid_f = jnp.where(d2_ref[...] <= R2, 1.0, 0.0)
    penal = (1.0 - valid_f) * NEG

    def mask_max(x3):
        return jnp.max(x3 * valid_f + penal, axis=1)

    def small_mm(W_ref, b):
        out = (relx * W_ref[0:1, :][None]
               + rely * W_ref[1:2, :][None]
               + relz * W_ref[2:3, :][None])
        return out + b[None]

    h3 = jax.nn.relu(small_mm(Wenc_ref, benc_ref[...]))
    renc = mask_max(h3)
    renc_ref[...] = renc

    relm = jnp.broadcast_to(renc[:, None, :], (CB, K, 32)).reshape(CB * K, 32)
    nbrf = nbrf_ref[...].reshape(CB * K, 128)
    f1a = small_mm(W1a_ref, b1_ref[...]).reshape(CB * K, 64)
    f1 = jax.nn.relu(
        f1a
        + jnp.dot(relm, W1b_ref[...], preferred_element_type=jnp.float32)
        + jnp.dot(nbrf, W1c_ref[...], preferred_element_type=jnp.float32))
    f2 = jax.nn.relu(
        jnp.dot(f1, W2_ref[...], preferred_element_type=jnp.float32)
        + b2_ref[...])
    f3 = jax.nn.relu(
        jnp.dot(f2, W3_ref[...], preferred_element_type=jnp.float32)
        + b3_ref[...])
    maxf_ref[...] = mask_max(f3.reshape(CB, K, 128))


def _edge(nbrx, nbry, nbrz, nbrf3, fpsp, d2sel, Wenc, benc, W1a, W1b, W1c,
          b1, W2, b2, W3, b3, interpret=False):
    grid = MP // CB
    full = lambda shape: pl.BlockSpec(shape, lambda g: tuple(0 for _ in shape))
    return pl.pallas_call(
        _edge_body,
        grid=(grid,),
        in_specs=[
            pl.BlockSpec((CB, K, 1), lambda g: (g, 0, 0)),
            pl.BlockSpec((CB, K, 1), lambda g: (g, 0, 0)),
            pl.BlockSpec((CB, K, 1), lambda g: (g, 0, 0)),
            pl.BlockSpec((CB, K, 128), lambda g: (g, 0, 0)),
            pl.BlockSpec((CB, 3, 1), lambda g: (g, 0, 0)),
            pl.BlockSpec((CB, K, 1), lambda g: (g, 0, 0)),
            full((3, 32)), full((1, 32)),
            full((3, 64)), full((32, 64)), full((128, 64)), full((1, 64)),
            full((64, 64)), full((1, 64)),
            full((64, 128)), full((1, 128)),
        ],
        out_specs=[
            pl.BlockSpec((CB, 32), lambda g: (g, 0)),
            pl.BlockSpec((CB, 128), lambda g: (g, 0)),
        ],
        out_shape=[
            jax.ShapeDtypeStruct((MP, 32), jnp.float32),
            jax.ShapeDtypeStruct((MP, 128), jnp.float32),
        ],
        interpret=interpret,
    )(nbrx, nbry, nbrz, nbrf3, fpsp, d2sel, Wenc, benc, W1a, W1b, W1c, b1,
      W2, b2, W3, b3)


def _gmlp_body(maxf_ref, Wg1_ref, bg1_ref, Wg2_ref, bg2_ref,
               Wg3_ref, bg3_ref, out_ref):
    g1 = jax.nn.relu(
        jnp.dot(maxf_ref[...], Wg1_ref[...],
                preferred_element_type=jnp.float32) + bg1_ref[...])
    g2 = jax.nn.relu(
        jnp.dot(g1, Wg2_ref[...], preferred_element_type=jnp.float32)
        + bg2_ref[...])
    out_ref[...] = jax.nn.relu(
        jnp.dot(g2, Wg3_ref[...], preferred_element_type=jnp.float32)
        + bg3_ref[...])


def _gmlp(maxf, Wg1, bg1, Wg2, bg2, Wg3, bg3, interpret=False):
    return pl.pallas_call(
        _gmlp_body,
        out_shape=jax.ShapeDtypeStruct((MP, 64), jnp.float32),
        interpret=interpret,
    )(maxf, Wg1, bg1, Wg2, bg2, Wg3, bg3)


@jax.jit
def _run(points, features, batch, W_enc, b_enc, W1, b1, W2, b2, W3, b3,
         Wg1, bg1, Wg2, bg2, Wg3, bg3):
    ptsT = points.T
    pad = ((0, 0), (0, FPS_LP - FPS_L))
    xs = jnp.pad(ptsT[0].reshape(FPS_S, FPS_L), pad, constant_values=jnp.inf)
    ys = jnp.pad(ptsT[1].reshape(FPS_S, FPS_L), pad, constant_values=jnp.inf)
    zs = jnp.pad(ptsT[2].reshape(FPS_S, FPS_L), pad, constant_values=jnp.inf)
    fps_idx = _fps(xs, ys, zs, ptsT[0], ptsT[1], ptsT[2])

    fps_idx_pad = jnp.concatenate(
        [fps_idx, jnp.zeros((MP - M,), jnp.int32)])
    fpsp_pad = points[fps_idx_pad]
    ptsT_pad = jnp.concatenate(
        [ptsT, jnp.full((3, NP - N), 1e15, jnp.float32)], axis=1)
    nbr_idx, d2sel = _topk(ptsT_pad, fpsp_pad)

    idx_flat = nbr_idx.reshape(-1)
    outx, outy, outz, outf = _sc_gather(
        points[:, 0], points[:, 1], points[:, 2], features, idx_flat)
    nbrx = outx.reshape(MP, K, 1)
    nbry = outy.reshape(MP, K, 1)
    nbrz = outz.reshape(MP, K, 1)
    nbrf3 = outf.reshape(MP, K, 128)

    renc, maxf = _edge(
        nbrx, nbry, nbrz, nbrf3, fpsp_pad.reshape(MP, 3, 1),
        d2sel.reshape(MP, K, 1), W_enc,
        b_enc.reshape(1, -1), W1[:3], W1[3:35], W1[35:],
        b1.reshape(1, -1), W2, b2.reshape(1, -1), W3, b3.reshape(1, -1))

    g3 = _gmlp(maxf, Wg1, bg1.reshape(1, -1), Wg2, bg2.reshape(1, -1),
               Wg3, bg3.reshape(1, -1))

    fps_points = fpsp_pad[:M]
    output_features = jnp.concatenate([renc[:M], g3[:M]], axis=1)
    fps_batch = batch[fps_idx]
    return (fps_points, output_features, fps_batch)


def kernel(points, features, batch, W_enc, b_enc, W1, b1, W2, b2, W3, b3,
           Wg1, bg1, Wg2, bg2, Wg3, bg3):
    return _run(points, features, batch, W_enc, b_enc, W1, b1, W2, b2,
                W3, b3, Wg1, bg1, Wg2, bg2, Wg3, bg3)
